# Initial kernel scaffold; baseline (speedup 1.0000x reference)
import jax
import jax.numpy as jnp
from jax.experimental import pallas as pl


def kernel(x, convW, convb, attW, attb):
    raise NotImplementedError("write your pallas kernel here")


# speedup vs baseline: 1.0495x; 1.0495x over previous
import functools

import jax
import jax.numpy as jnp
from jax.experimental import pallas as pl
from jax.experimental.pallas import tpu as pltpu

_N_NODES = 8
_OUT_NEIS = [list(range(i + 1, min(i + 4, _N_NODES))) for i in range(_N_NODES - 1)]
_N_IN = [0] * _N_NODES
for _i, _outs in enumerate(_OUT_NEIS):
    for _t in _outs:
        _N_IN[_t] += 1
_TOPK = 4
_EPS = 0.01

_H = 28
_W = 28
_HP = 31
_WP = 32
_SPAD = _HP * _WP
_SWIN = _H * _WP
_SOUT = _WP + 1
_NPIX = float(_H * _W)


def _conv_body(n_in, C, g_ref, *refs):
    f_refs = refs[:n_in]
    w_ref = refs[n_in]
    b_ref = refs[n_in + 1]
    feat_out = refs[n_in + 2]
    pool_out = refs[n_in + 3]

    acc = g_ref[0, 0, 0] * f_refs[0][0]
    for j in range(1, n_in):
        acc = acc + g_ref[0, 0, j] * f_refs[j][0]

    patches = jnp.concatenate(
        [acc[:, dy * _WP + dx: dy * _WP + dx + _SWIN]
         for dy in range(3) for dx in range(3)], axis=0)
    out = jnp.dot(w_ref[...], patches, preferred_element_type=jnp.float32)
    out = jnp.maximum(out + b_ref[...], 0.0)

    lane = jax.lax.broadcasted_iota(jnp.int32, (1, _SWIN), 1)
    colmask = ((lane % _WP) < _W).astype(jnp.float32)
    out = out * colmask

    pool_out[0, 0, :] = jnp.sum(out, axis=1) * (1.0 / _NPIX)
    feat_out[0] = jnp.zeros((C, _SPAD), jnp.float32)
    feat_out[0, :, _SOUT:_SOUT + _SWIN] = out


def _conv_call(n_in, feats, gates, wmat, bvec):
    B, C, _ = feats[0].shape
    in_specs = (
        [pl.BlockSpec((1, 1, n_in), lambda b: (b, 0, 0))]
        + [pl.BlockSpec((1, C, _SPAD), lambda b: (b, 0, 0))] * n_in
        + [pl.BlockSpec((C, 9 * C), lambda b: (0, 0)),
           pl.BlockSpec((C, 1), lambda b: (0, 0))]
    )
    out_specs = [pl.BlockSpec((1, C, _SPAD), lambda b: (b, 0, 0)),
                 pl.BlockSpec((1, 1, C), lambda b: (b, 0, 0))]
    out_shape = [jax.ShapeDtypeStruct((B, C, _SPAD), jnp.float32),
                 jax.ShapeDtypeStruct((B, 1, C), jnp.float32)]
    return pl.pallas_call(
        functools.partial(_conv_body, n_in, C),
        grid=(B,),
        in_specs=in_specs,
        out_specs=out_specs,
        out_shape=out_shape,
        compiler_params=pltpu.CompilerParams(
            dimension_semantics=("parallel",)),
    )(gates, *feats, wmat, bvec)


def _route_body(i, n_out, B, *refs):
    outs = _OUT_NEIS[i]
    if n_out > 1:
        pooled_ref, w_ref, b_ref, ad_ref, na_ref = refs[:5]
        ad_out, na_out, g_out = refs[5:]
        scores = jnp.dot(pooled_ref[...], w_ref[...],
                         preferred_element_type=jnp.float32) + b_ref[...]
        m = jnp.max(scores, axis=1, keepdims=True)
        e = jnp.exp(scores - m)
        p = e / jnp.sum(e, axis=1, keepdims=True)
    else:
        ad_ref, na_ref = refs[:2]
        ad_out, na_out, g_out = refs[2:]
        p = jnp.ones((B, 1), jnp.float32)

    ad = ad_ref[...]
    na = na_ref[...]
    a_sent = na[:, i:i + 1] * p

    cols = []
    for t in range(_N_NODES):
        if t == i:
            cols.append(jnp.zeros((B, 1), jnp.float32))
        elif t in outs:
            cols.append(ad[:, t:t + 1] + a_sent[:, outs.index(t):outs.index(t) + 1])
        else:
            cols.append(ad[:, t:t + 1])
    ad = jnp.concatenate(cols, axis=1)

    cnt_cols = []
    for j in range(_N_NODES):
        c = jnp.zeros((B, 1), jnp.float32)
        for k in range(_N_NODES):
            if k != j:
                c = c + (ad[:, k:k + 1] > ad[:, j:j + 1]).astype(jnp.float32)
        cnt_cols.append(c)
    cnt = jnp.concatenate(cnt_cols, axis=1)
    cand = jnp.where(cnt <= float(_TOPK - 1), ad, jnp.float32(1e30))
    thresh = jnp.min(cand, axis=1, keepdims=True)

    mask = jnp.logical_and(ad >= thresh, ad > _EPS).astype(jnp.float32)
    masked = ad * mask
    scale = 1.0 / (jnp.sum(masked, axis=1, keepdims=True) + 1e-6)
    na_out[...] = masked * scale
    ad_out[...] = masked

    g = a_sent * jnp.concatenate(
        [mask[:, t:t + 1] for t in outs], axis=1) * scale
    if n_out < 3:
        g = jnp.concatenate(
            [g, jnp.zeros((B, 3 - n_out), jnp.float32)], axis=1)
    g_out[...] = g


def _route_call(i, pooled, attW, attb, ad, na):
    B = ad.shape[0]
    n_out = len(_OUT_NEIS[i])
    out_shape = [jax.ShapeDtypeStruct((B, _N_NODES), jnp.float32),
                 jax.ShapeDtypeStruct((B, _N_NODES), jnp.float32),
                 jax.ShapeDtypeStruct((B, 3), jnp.float32)]
    body = functools.partial(_route_body, i, n_out, B)
    if n_out > 1:
        args = (pooled, attW[i][:, :n_out], attb[i][:n_out].reshape(1, n_out),
                ad, na)
    else:
        args = (ad, na)
    return pl.pallas_call(body, out_shape=out_shape)(*args)


def _pool_body(x_ref, out_ref):
    out_ref[0, 0, :] = jnp.sum(x_ref[0], axis=1) * (1.0 / _NPIX)


def _final_body(g_ref, f1_ref, f2_ref, f3_ref, out_ref):
    out_ref[0] = (g_ref[0, 0, 0] * f1_ref[0] + g_ref[0, 0, 1] * f2_ref[0]
                  + g_ref[0, 0, 2] * f3_ref[0])


def kernel(x, convW, convb, attW, attb):
    B, C, H, W = x.shape
    xl = jnp.pad(x, ((0, 0), (0, 0), (1, _HP - H - 1), (1, _WP - W - 1))
                 ).reshape(B, C, _SPAD)

    wall = jnp.transpose(convW.reshape(6, 9, C, C), (0, 3, 1, 2)
                         ).reshape(6, C, 9 * C)

    pooled = pl.pallas_call(
        _pool_body,
        grid=(B,),
        in_specs=[pl.BlockSpec((1, C, _SPAD), lambda b: (b, 0, 0))],
        out_specs=pl.BlockSpec((1, 1, C), lambda b: (b, 0, 0)),
        out_shape=jax.ShapeDtypeStruct((B, 1, C), jnp.float32),
        compiler_params=pltpu.CompilerParams(
            dimension_semantics=("parallel",)),
    )(xl).reshape(B, C)

    ad = jnp.zeros((B, _N_NODES), jnp.float32).at[:, 0].set(1.0)
    na = ad
    feats = {0: xl}
    gates = {}
    ad, na, gates[0] = _route_call(0, pooled, attW, attb, ad, na)

    for t in range(1, _N_NODES - 1):
        ins = [i for i in range(max(0, t - 3), t) if t in _OUT_NEIS[i]]
        gs = (jnp.stack([gates[i][:, t - i - 1] for i in ins], axis=1)
              * (1.0 / _N_IN[t])).reshape(B, 1, len(ins))
        feat, pooled = _conv_call(
            len(ins), [feats[i] for i in ins], gs,
            wall[t - 1], convb[t - 1].reshape(C, 1))
        feats[t] = feat
        pooled = pooled.reshape(B, C)
        ad, na, gates[t] = _route_call(t, pooled, attW, attb, ad, na)

    gs7 = (jnp.stack([gates[i][:, _N_NODES - 1 - i - 1] for i in (4, 5, 6)],
                     axis=1) * (1.0 / _N_IN[_N_NODES - 1])).reshape(B, 1, 3)
    out = pl.pallas_call(
        _final_body,
        grid=(B,),
        in_specs=[pl.BlockSpec((1, 1, 3), lambda b: (b, 0, 0))]
        + [pl.BlockSpec((1, C, _SPAD), lambda b: (b, 0, 0))] * 3,
        out_specs=pl.BlockSpec((1, C, _SPAD), lambda b: (b, 0, 0)),
        out_shape=jax.ShapeDtypeStruct((B, C, _SPAD), jnp.float32),
        compiler_params=pltpu.CompilerParams(
            dimension_semantics=("parallel",)),
    )(gs7, feats[4], feats[5], feats[6])
    return out.reshape(B, C, _HP, _WP)[:, :, 1:_H + 1, 1:_W + 1]


# speedup vs baseline: 1.1993x; 1.1428x over previous
import jax
import jax.numpy as jnp
from jax.experimental import pallas as pl
from jax.experimental.pallas import tpu as pltpu

_N_NODES = 8
_OUT_NEIS = [list(range(i + 1, min(i + 4, _N_NODES))) for i in range(_N_NODES - 1)]
_TOPK = 4
_EPS = 0.01

_B = 32
_C = 96
_NCORES = 2
_NB = _B // _NCORES

_H = 28
_W = 28
_HP = 31
_WP = 32
_SPAD = _HP * _WP
_SWIN = _H * _WP
_SOUT = _WP + 1
_NPIX = float(_H * _W)


def _onehot(i):
    lane = jax.lax.broadcasted_iota(jnp.int32, (1, _NB), 1)
    return (lane == i).astype(jnp.float32)


def _gather(s, i, featbuf, gscr, nin_inv):
    oh = _onehot(i)
    acc = jnp.zeros((_C, _SPAD), jnp.float32)
    for d in (1, 2, 3):
        isrc = s - d
        valid = isrc >= 0
        isc = jnp.maximum(isrc, 0)
        slot = jax.lax.rem(isc, 4)
        grow = gscr[pl.ds(isc, 1), d - 1]
        g = jnp.sum(grow * oh, axis=1, keepdims=True)
        g = jnp.where(valid, g, 0.0)
        feat = featbuf[pl.ds(slot, 1), pl.ds(i, 1)][0, 0]
        acc = acc + g * feat
    return acc * nin_inv


def _fused_body(x_ref, w_ref, b_ref, aw_ref, ab_ref, out_ref,
                featbuf, pooled, ad_s, na_s, gscr):
    s = pl.program_id(1)
    i = pl.program_id(2)

    def _pool_store(vec_col):
        base = jnp.where(i == 0, jnp.zeros((_C, _NB), jnp.float32),
                         pooled[...])
        pooled[...] = base + vec_col * _onehot(i)

    @pl.when(jnp.logical_and(s == 0, i < _NB))
    def _():
        xb = x_ref[0]
        featbuf[0, pl.ds(i, 1)] = xb[None]
        _pool_store(jnp.sum(xb, axis=1)[:, None] * (1.0 / _NPIX))

    @pl.when(jnp.logical_and(jnp.logical_and(s >= 1, s <= 6), i < _NB))
    def _():
        nin_inv = jnp.where(s == 1, 1.0, jnp.where(s == 2, 0.5, 1.0 / 3.0))
        acc = _gather(s, i, featbuf, gscr, nin_inv)
        patches = jnp.concatenate(
            [acc[:, dy * _WP + dx: dy * _WP + dx + _SWIN]
             for dy in range(3) for dx in range(3)], axis=0)
        wm = w_ref[pl.ds(s - 1, 1)][0]
        out = jnp.dot(wm, patches, preferred_element_type=jnp.float32)
        bb = b_ref[pl.ds(s - 1, 1)][0]
        out = jnp.maximum(out + bb, 0.0)
        lane = jax.lax.broadcasted_iota(jnp.int32, (1, _SWIN), 1)
        out = out * ((lane % _WP) < _W).astype(jnp.float32)
        _pool_store(jnp.sum(out, axis=1)[:, None] * (1.0 / _NPIX))
        slw = jax.lax.rem(s, 4)
        featbuf[pl.ds(slw, 1), pl.ds(i, 1), :, 0:_SOUT] = \
            jnp.zeros((1, 1, _C, _SOUT), jnp.float32)
        featbuf[pl.ds(slw, 1), pl.ds(i, 1), :, _SOUT:_SOUT + _SWIN] = \
            out[None, None]
        featbuf[pl.ds(slw, 1), pl.ds(i, 1), :, _SOUT + _SWIN:] = \
            jnp.zeros((1, 1, _C, _SPAD - _SOUT - _SWIN), jnp.float32)

    @pl.when(jnp.logical_and(s == 7, i < _NB))
    def _():
        out_ref[0] = _gather(s, i, featbuf, gscr, 1.0 / 3.0)

    for sc in range(_N_NODES - 1):
        @pl.when(jnp.logical_and(s == sc, i == _NB))
        def _(sc=sc):
            outs = _OUT_NEIS[sc]
            n_out = len(outs)
            if sc == 0:
                ad = jnp.concatenate(
                    [jnp.ones((1, _NB), jnp.float32),
                     jnp.zeros((_N_NODES - 1, _NB), jnp.float32)], axis=0)
                na = ad
            else:
                ad = ad_s[...]
                na = na_s[...]
            if n_out > 1:
                sco = jnp.dot(aw_ref[sc, :n_out], pooled[...],
                              preferred_element_type=jnp.float32) \
                    + ab_ref[sc, :n_out]
                m = jnp.max(sco, axis=0, keepdims=True)
                e = jnp.exp(sco - m)
                p = e / jnp.sum(e, axis=0, keepdims=True)
            else:
                p = jnp.ones((1, _NB), jnp.float32)
            a_sent = na[sc:sc + 1, :] * p

            rows = []
            for t in range(_N_NODES):
                if t == sc:
                    rows.append(jnp.zeros((1, _NB), jnp.float32))
                elif t in outs:
                    j = outs.index(t)
                    rows.append(ad[t:t + 1] + a_sent[j:j + 1])
                else:
                    rows.append(ad[t:t + 1])
            ad2 = jnp.concatenate(rows, axis=0)

            cnt_rows = []
            for j in range(_N_NODES):
                c = jnp.zeros((1, _NB), jnp.float32)
                for k in range(_N_NODES):
                    if k != j:
                        c = c + (ad2[k:k + 1] > ad2[j:j + 1]).astype(jnp.float32)
                cnt_rows.append(c)
            cnt = jnp.concatenate(cnt_rows, axis=0)
            cand = jnp.where(cnt <= float(_TOPK - 1), ad2, jnp.float32(1e30))
            thresh = jnp.min(cand, axis=0, keepdims=True)

            mask = jnp.logical_and(ad2 >= thresh, ad2 > _EPS).astype(jnp.float32)
            masked = ad2 * mask
            scale = 1.0 / (jnp.sum(masked, axis=0, keepdims=True) + 1e-6)
            na_s[...] = masked * scale
            ad_s[...] = masked

            grows = [a_sent[j:j + 1] * mask[t:t + 1] * scale
                     for j, t in enumerate(outs)]
            grows += [jnp.zeros((1, _NB), jnp.float32)] * (3 - n_out)
            gscr[sc] = jnp.concatenate(grows, axis=0)


def kernel(x, convW, convb, attW, attb):
    B, C, H, W = x.shape
    xl = jnp.pad(x, ((0, 0), (0, 0), (1, _HP - H - 1), (1, _WP - W - 1))
                 ).reshape(B, C, _SPAD)
    wall = jnp.transpose(convW.reshape(6, 9, C, C), (0, 3, 1, 2)
                         ).reshape(6, C, 9 * C)
    bcol = convb.reshape(6, C, 1)
    awT = jnp.transpose(attW, (0, 2, 1))
    abT = attb.reshape(6, 3, 1)

    out = pl.pallas_call(
        _fused_body,
        grid=(_NCORES, _N_NODES, _NB + 1),
        in_specs=[
            pl.BlockSpec(
                (1, C, _SPAD),
                lambda p, s, i: (p * _NB + jnp.where(
                    s == 0, jnp.minimum(i, _NB - 1), _NB - 1), 0, 0)),
            pl.BlockSpec((6, C, 9 * C), lambda p, s, i: (0, 0, 0)),
            pl.BlockSpec((6, C, 1), lambda p, s, i: (0, 0, 0)),
            pl.BlockSpec((6, 3, C), lambda p, s, i: (0, 0, 0)),
            pl.BlockSpec((6, 3, 1), lambda p, s, i: (0, 0, 0)),
        ],
        out_specs=pl.BlockSpec(
            (1, C, _SPAD),
            lambda p, s, i: (p * _NB + jnp.where(
                s == 7, jnp.minimum(i, _NB - 1), 0), 0, 0)),
        out_shape=jax.ShapeDtypeStruct((B, C, _SPAD), jnp.float32),
        scratch_shapes=[
            pltpu.VMEM((4, _NB, C, _SPAD), jnp.float32),
            pltpu.VMEM((C, _NB), jnp.float32),
            pltpu.VMEM((_N_NODES, _NB), jnp.float32),
            pltpu.VMEM((_N_NODES, _NB), jnp.float32),
            pltpu.VMEM((_N_NODES - 1, 3, _NB), jnp.float32),
        ],
        compiler_params=pltpu.CompilerParams(
            dimension_semantics=("parallel", "arbitrary", "arbitrary")),
    )(xl, wall, bcol, awT, abT)
    return out.reshape(B, C, _HP, _WP)[:, :, 1:_H + 1, 1:_W + 1]


# speedup vs baseline: 1.4357x; 1.1971x over previous
import jax
import jax.numpy as jnp
from jax.experimental import pallas as pl
from jax.experimental.pallas import tpu as pltpu

_N_NODES = 8
_OUT_NEIS = [list(range(i + 1, min(i + 4, _N_NODES))) for i in range(_N_NODES - 1)]
_TOPK = 4
_EPS = 0.01

_B = 32
_C = 96
_NCORES = 2
_NB = _B // _NCORES
_PAIRS = _NB // 2

_H = 28
_W = 28
_HP = 31
_WP = 32
_SPAD = _HP * _WP
_SWIN = _H * _WP
_SOUT = _WP + 1
_NPIX = float(_H * _W)


def _onehot(i):
    lane = jax.lax.broadcasted_iota(jnp.int32, (1, _NB), 1)
    return (lane == i).astype(jnp.float32)


def _gather(s, i, featbuf, gscr, nin_inv):
    oh = _onehot(i)
    acc = jnp.zeros((_C, _SPAD), jnp.float32)
    for d in (1, 2, 3):
        isrc = s - d
        valid = isrc >= 0
        isc = jnp.maximum(isrc, 0)
        slot = jax.lax.rem(isc, 4)
        grow = gscr[pl.ds(isc, 1), d - 1]
        g = jnp.sum(grow * oh, axis=1, keepdims=True)
        g = jnp.where(valid, g, 0.0)
        feat = featbuf[pl.ds(slot, 1), pl.ds(i, 1)][0, 0]
        acc = acc + g * feat
    return acc * nin_inv


def _fused_body(x_ref, w_ref, b_ref, aw_ref, ab_ref, out_ref,
                featbuf, pooled, ad_s, na_s, gscr):
    s = pl.program_id(1)
    i = pl.program_id(2)

    def _pool_store2(v0, v1):
        base = jnp.where(i == 0, jnp.zeros((_C, _NB), jnp.float32),
                         pooled[...])
        pooled[...] = base + v0 * _onehot(2 * i) + v1 * _onehot(2 * i + 1)

    @pl.when(jnp.logical_and(s == 0, i < _PAIRS))
    def _():
        xb = x_ref[...]
        featbuf[0, pl.ds(2 * i, 2)] = xb
        _pool_store2(jnp.sum(xb[0], axis=1)[:, None] * (1.0 / _NPIX),
                     jnp.sum(xb[1], axis=1)[:, None] * (1.0 / _NPIX))

    @pl.when(jnp.logical_and(jnp.logical_and(s >= 1, s <= 6), i < _PAIRS))
    def _():
        nin_inv = jnp.where(s == 1, 1.0, jnp.where(s == 2, 0.5, 1.0 / 3.0))
        acc0 = _gather(s, 2 * i, featbuf, gscr, nin_inv)
        acc1 = _gather(s, 2 * i + 1, featbuf, gscr, nin_inv)
        patches = jnp.concatenate(
            [jnp.concatenate(
                [acc0[:, dy * _WP + dx: dy * _WP + dx + _SWIN],
                 acc1[:, dy * _WP + dx: dy * _WP + dx + _SWIN]], axis=1)
             for dy in range(3) for dx in range(3)], axis=0)
        wm = w_ref[pl.ds(s - 1, 1)][0]
        out = jnp.dot(wm, patches, preferred_element_type=jnp.float32)
        bb = b_ref[pl.ds(s - 1, 1)][0]
        out = jnp.maximum(out + bb, 0.0)
        lane = jax.lax.broadcasted_iota(jnp.int32, (1, 2 * _SWIN), 1)
        out = out * ((lane % _WP) < _W).astype(jnp.float32)
        o0 = out[:, :_SWIN]
        o1 = out[:, _SWIN:]
        _pool_store2(jnp.sum(o0, axis=1)[:, None] * (1.0 / _NPIX),
                     jnp.sum(o1, axis=1)[:, None] * (1.0 / _NPIX))
        slw = jax.lax.rem(s, 4)
        for b_off, ob in ((0, o0), (1, o1)):
            featbuf[pl.ds(slw, 1), pl.ds(2 * i + b_off, 1), :, 0:_SOUT] = \
                jnp.zeros((1, 1, _C, _SOUT), jnp.float32)
            featbuf[pl.ds(slw, 1), pl.ds(2 * i + b_off, 1), :,
                    _SOUT:_SOUT + _SWIN] = ob[None, None]
            featbuf[pl.ds(slw, 1), pl.ds(2 * i + b_off, 1), :,
                    _SOUT + _SWIN:] = \
                jnp.zeros((1, 1, _C, _SPAD - _SOUT - _SWIN), jnp.float32)

    @pl.when(jnp.logical_and(s == 7, i < _PAIRS))
    def _():
        out_ref[0] = _gather(s, 2 * i, featbuf, gscr, 1.0 / 3.0)
        out_ref[1] = _gather(s, 2 * i + 1, featbuf, gscr, 1.0 / 3.0)

    for sc in range(_N_NODES - 1):
        @pl.when(jnp.logical_and(s == sc, i == _PAIRS))
        def _(sc=sc):
            outs = _OUT_NEIS[sc]
            n_out = len(outs)
            if sc == 0:
                ad = jnp.concatenate(
                    [jnp.ones((1, _NB), jnp.float32),
                     jnp.zeros((_N_NODES - 1, _NB), jnp.float32)], axis=0)
                na = ad
            else:
                ad = ad_s[...]
                na = na_s[...]
            if n_out > 1:
                sco = jnp.dot(aw_ref[sc, :n_out], pooled[...],
                              preferred_element_type=jnp.float32) \
                    + ab_ref[sc, :n_out]
                m = jnp.max(sco, axis=0, keepdims=True)
                e = jnp.exp(sco - m)
                p = e / jnp.sum(e, axis=0, keepdims=True)
            else:
                p = jnp.ones((1, _NB), jnp.float32)
            a_sent = na[sc:sc + 1, :] * p

            rows = []
            for t in range(_N_NODES):
                if t == sc:
                    rows.append(jnp.zeros((1, _NB), jnp.float32))
                elif t in outs:
                    j = outs.index(t)
                    rows.append(ad[t:t + 1] + a_sent[j:j + 1])
                else:
                    rows.append(ad[t:t + 1])
            ad2 = jnp.concatenate(rows, axis=0)

            cnt_rows = []
            for j in range(_N_NODES):
                c = jnp.zeros((1, _NB), jnp.float32)
                for k in range(_N_NODES):
                    if k != j:
                        c = c + (ad2[k:k + 1] > ad2[j:j + 1]).astype(jnp.float32)
                cnt_rows.append(c)
            cnt = jnp.concatenate(cnt_rows, axis=0)
            cand = jnp.where(cnt <= float(_TOPK - 1), ad2, jnp.float32(1e30))
            thresh = jnp.min(cand, axis=0, keepdims=True)

            mask = jnp.logical_and(ad2 >= thresh, ad2 > _EPS).astype(jnp.float32)
            masked = ad2 * mask
            scale = 1.0 / (jnp.sum(masked, axis=0, keepdims=True) + 1e-6)
            na_s[...] = masked * scale
            ad_s[...] = masked

            grows = [a_sent[j:j + 1] * mask[t:t + 1] * scale
                     for j, t in enumerate(outs)]
            grows += [jnp.zeros((1, _NB), jnp.float32)] * (3 - n_out)
            gscr[sc] = jnp.concatenate(grows, axis=0)


def kernel(x, convW, convb, attW, attb):
    B, C, H, W = x.shape
    xl = jnp.pad(x, ((0, 0), (0, 0), (1, _HP - H - 1), (1, _WP - W - 1))
                 ).reshape(B, C, _SPAD)
    wall = jnp.transpose(convW.reshape(6, 9, C, C), (0, 3, 1, 2)
                         ).reshape(6, C, 9 * C)
    bcol = convb.reshape(6, C, 1)
    awT = jnp.transpose(attW, (0, 2, 1))
    abT = attb.reshape(6, 3, 1)

    out = pl.pallas_call(
        _fused_body,
        grid=(_NCORES, _N_NODES, _PAIRS + 1),
        in_specs=[
            pl.BlockSpec(
                (2, C, _SPAD),
                lambda p, s, i: (p * _PAIRS + jnp.where(
                    s == 0, jnp.minimum(i, _PAIRS - 1), _PAIRS - 1), 0, 0)),
            pl.BlockSpec((6, C, 9 * C), lambda p, s, i: (0, 0, 0)),
            pl.BlockSpec((6, C, 1), lambda p, s, i: (0, 0, 0)),
            pl.BlockSpec((6, 3, C), lambda p, s, i: (0, 0, 0)),
            pl.BlockSpec((6, 3, 1), lambda p, s, i: (0, 0, 0)),
        ],
        out_specs=pl.BlockSpec(
            (2, C, _SPAD),
            lambda p, s, i: (p * _PAIRS + jnp.where(
                s == 7, jnp.minimum(i, _PAIRS - 1), 0), 0, 0)),
        out_shape=jax.ShapeDtypeStruct((B, C, _SPAD), jnp.float32),
        scratch_shapes=[
            pltpu.VMEM((4, _NB, C, _SPAD), jnp.float32),
            pltpu.VMEM((C, _NB), jnp.float32),
            pltpu.VMEM((_N_NODES, _NB), jnp.float32),
            pltpu.VMEM((_N_NODES, _NB), jnp.float32),
            pltpu.VMEM((_N_NODES - 1, 3, _NB), jnp.float32),
        ],
        compiler_params=pltpu.CompilerParams(
            dimension_semantics=("parallel", "arbitrary", "arbitrary")),
    )(xl, wall, bcol, awT, abT)
    return out.reshape(B, C, _HP, _WP)[:, :, 1:_H + 1, 1:_W + 1]


# speedup vs baseline: 1.5488x; 1.0788x over previous
import jax
import jax.numpy as jnp
from jax.experimental import pallas as pl
from jax.experimental.pallas import tpu as pltpu

_N_NODES = 8
_OUT_NEIS = [list(range(i + 1, min(i + 4, _N_NODES))) for i in range(_N_NODES - 1)]
_TOPK = 4
_EPS = 0.01

_B = 32
_C = 96
_NCORES = 2
_NB = _B // _NCORES
_IPG = 4
_PAIRS = _NB // _IPG

_H = 28
_W = 28
_HP = 31
_WP = 32
_SPAD = _HP * _WP
_SWIN = _H * _WP
_SOUT = _WP + 1
_NPIX = float(_H * _W)


def _onehot(i):
    lane = jax.lax.broadcasted_iota(jnp.int32, (1, _NB), 1)
    return (lane == i).astype(jnp.float32)


def _gather(s, i, featbuf, gscr, nin_inv):
    oh = _onehot(i)
    acc = jnp.zeros((_C, _SPAD), jnp.float32)
    for d in (1, 2, 3):
        isrc = s - d
        valid = isrc >= 0
        isc = jnp.maximum(isrc, 0)
        slot = jax.lax.rem(isc, 4)
        grow = gscr[pl.ds(isc, 1), d - 1]
        g = jnp.sum(grow * oh, axis=1, keepdims=True)
        g = jnp.where(valid, g, 0.0)
        feat = featbuf[pl.ds(slot, 1), pl.ds(i, 1)][0, 0]
        acc = acc + g * feat
    return acc * nin_inv


def _fused_body(x_ref, w_ref, b_ref, aw_ref, ab_ref, out_ref,
                featbuf, pooled, ad_s, na_s, gscr):
    s = pl.program_id(1)
    i = pl.program_id(2)

    def _pool_store(vecs):
        base = jnp.where(i == 0, jnp.zeros((_C, _NB), jnp.float32),
                         pooled[...])
        for g, v in enumerate(vecs):
            base = base + v * _onehot(_IPG * i + g)
        pooled[...] = base

    @pl.when(jnp.logical_and(s == 0, i < _PAIRS))
    def _():
        xb = x_ref[...]
        featbuf[0, pl.ds(_IPG * i, _IPG)] = xb
        _pool_store([jnp.sum(xb[g], axis=1)[:, None] * (1.0 / _NPIX)
                     for g in range(_IPG)])

    @pl.when(jnp.logical_and(jnp.logical_and(s >= 1, s <= 6), i < _PAIRS))
    def _():
        nin_inv = jnp.where(s == 1, 1.0, jnp.where(s == 2, 0.5, 1.0 / 3.0))
        accs = [_gather(s, _IPG * i + g, featbuf, gscr, nin_inv)
                for g in range(_IPG)]
        patches = jnp.concatenate(
            [jnp.concatenate(
                [a[:, dy * _WP + dx: dy * _WP + dx + _SWIN] for a in accs],
                axis=1)
             for dy in range(3) for dx in range(3)], axis=0)
        wm = w_ref[pl.ds(s - 1, 1)][0]
        out = jnp.dot(wm, patches, preferred_element_type=jnp.float32)
        bb = b_ref[pl.ds(s - 1, 1)][0]
        out = jnp.maximum(out + bb, 0.0)
        lane = jax.lax.broadcasted_iota(jnp.int32, (1, _IPG * _SWIN), 1)
        out = out * ((lane % _WP) < _W).astype(jnp.float32)
        os = [out[:, g * _SWIN:(g + 1) * _SWIN] for g in range(_IPG)]
        _pool_store([jnp.sum(o, axis=1)[:, None] * (1.0 / _NPIX) for o in os])
        slw = jax.lax.rem(s, 4)
        for g, ob in enumerate(os):
            featbuf[pl.ds(slw, 1), pl.ds(_IPG * i + g, 1), :, 0:_SOUT] = \
                jnp.zeros((1, 1, _C, _SOUT), jnp.float32)
            featbuf[pl.ds(slw, 1), pl.ds(_IPG * i + g, 1), :,
                    _SOUT:_SOUT + _SWIN] = ob[None, None]
            featbuf[pl.ds(slw, 1), pl.ds(_IPG * i + g, 1), :,
                    _SOUT + _SWIN:] = \
                jnp.zeros((1, 1, _C, _SPAD - _SOUT - _SWIN), jnp.float32)

    @pl.when(jnp.logical_and(s == 7, i < _PAIRS))
    def _():
        for g in range(_IPG):
            out_ref[g] = _gather(s, _IPG * i + g, featbuf, gscr, 1.0 / 3.0)

    for sc in range(_N_NODES - 1):
        @pl.when(jnp.logical_and(s == sc, i == _PAIRS))
        def _(sc=sc):
            outs = _OUT_NEIS[sc]
            n_out = len(outs)
            if sc == 0:
                ad = jnp.concatenate(
                    [jnp.ones((1, _NB), jnp.float32),
                     jnp.zeros((_N_NODES - 1, _NB), jnp.float32)], axis=0)
                na = ad
            else:
                ad = ad_s[...]
                na = na_s[...]
            if n_out > 1:
                sco = jnp.dot(aw_ref[sc, :n_out], pooled[...],
                              preferred_element_type=jnp.float32) \
                    + ab_ref[sc, :n_out]
                m = jnp.max(sco, axis=0, keepdims=True)
                e = jnp.exp(sco - m)
                p = e / jnp.sum(e, axis=0, keepdims=True)
            else:
                p = jnp.ones((1, _NB), jnp.float32)
            a_sent = na[sc:sc + 1, :] * p

            rows = []
            for t in range(_N_NODES):
                if t == sc:
                    rows.append(jnp.zeros((1, _NB), jnp.float32))
                elif t in outs:
                    j = outs.index(t)
                    rows.append(ad[t:t + 1] + a_sent[j:j + 1])
                else:
                    rows.append(ad[t:t + 1])
            ad2 = jnp.concatenate(rows, axis=0)

            cnt_rows = []
            for j in range(_N_NODES):
                c = jnp.zeros((1, _NB), jnp.float32)
                for k in range(_N_NODES):
                    if k != j:
                        c = c + (ad2[k:k + 1] > ad2[j:j + 1]).astype(jnp.float32)
                cnt_rows.append(c)
            cnt = jnp.concatenate(cnt_rows, axis=0)
            cand = jnp.where(cnt <= float(_TOPK - 1), ad2, jnp.float32(1e30))
            thresh = jnp.min(cand, axis=0, keepdims=True)

            mask = jnp.logical_and(ad2 >= thresh, ad2 > _EPS).astype(jnp.float32)
            masked = ad2 * mask
            scale = 1.0 / (jnp.sum(masked, axis=0, keepdims=True) + 1e-6)
            na_s[...] = masked * scale
            ad_s[...] = masked

            grows = [a_sent[j:j + 1] * mask[t:t + 1] * scale
                     for j, t in enumerate(outs)]
            grows += [jnp.zeros((1, _NB), jnp.float32)] * (3 - n_out)
            gscr[sc] = jnp.concatenate(grows, axis=0)


def kernel(x, convW, convb, attW, attb):
    B, C, H, W = x.shape
    xl = jnp.pad(x, ((0, 0), (0, 0), (1, _HP - H - 1), (1, _WP - W - 1))
                 ).reshape(B, C, _SPAD)
    wall = jnp.transpose(convW.reshape(6, 9, C, C), (0, 3, 1, 2)
                         ).reshape(6, C, 9 * C)
    bcol = convb.reshape(6, C, 1)
    awT = jnp.transpose(attW, (0, 2, 1))
    abT = attb.reshape(6, 3, 1)

    out = pl.pallas_call(
        _fused_body,
        grid=(_NCORES, _N_NODES, _PAIRS + 1),
        in_specs=[
            pl.BlockSpec(
                (_IPG, C, _SPAD),
                lambda p, s, i: (p * _PAIRS + jnp.where(
                    s == 0, jnp.minimum(i, _PAIRS - 1), _PAIRS - 1), 0, 0)),
            pl.BlockSpec((6, C, 9 * C), lambda p, s, i: (0, 0, 0)),
            pl.BlockSpec((6, C, 1), lambda p, s, i: (0, 0, 0)),
            pl.BlockSpec((6, 3, C), lambda p, s, i: (0, 0, 0)),
            pl.BlockSpec((6, 3, 1), lambda p, s, i: (0, 0, 0)),
        ],
        out_specs=pl.BlockSpec(
            (_IPG, C, _SPAD),
            lambda p, s, i: (p * _PAIRS + jnp.where(
                s == 7, jnp.minimum(i, _PAIRS - 1), 0), 0, 0)),
        out_shape=jax.ShapeDtypeStruct((B, C, _SPAD), jnp.float32),
        scratch_shapes=[
            pltpu.VMEM((4, _NB, C, _SPAD), jnp.float32),
            pltpu.VMEM((C, _NB), jnp.float32),
            pltpu.VMEM((_N_NODES, _NB), jnp.float32),
            pltpu.VMEM((_N_NODES, _NB), jnp.float32),
            pltpu.VMEM((_N_NODES - 1, 3, _NB), jnp.float32),
        ],
        compiler_params=pltpu.CompilerParams(
            dimension_semantics=("parallel", "arbitrary", "arbitrary")),
    )(xl, wall, bcol, awT, abT)
    return out.reshape(B, C, _HP, _WP)[:, :, 1:_H + 1, 1:_W + 1]


# speedup vs baseline: 1.6095x; 1.0392x over previous
import jax
import jax.numpy as jnp
from jax.experimental import pallas as pl
from jax.experimental.pallas import tpu as pltpu

_N_NODES = 8
_OUT_NEIS = [list(range(i + 1, min(i + 4, _N_NODES))) for i in range(_N_NODES - 1)]
_TOPK = 4
_EPS = 0.01

_B = 32
_C = 96
_NCORES = 2
_NB = _B // _NCORES
_IPG = 8
_PAIRS = _NB // _IPG

_H = 28
_W = 28
_HP = 31
_WP = 32
_SPAD = _HP * _WP
_SWIN = _H * _WP
_SOUT = _WP + 1
_NPIX = float(_H * _W)


def _onehot(i):
    lane = jax.lax.broadcasted_iota(jnp.int32, (1, _NB), 1)
    return (lane == i).astype(jnp.float32)


def _gather(s, i, featbuf, gscr, nin_inv):
    oh = _onehot(i)
    acc = jnp.zeros((_C, _SPAD), jnp.float32)
    for d in (1, 2, 3):
        isrc = s - d
        valid = isrc >= 0
        isc = jnp.maximum(isrc, 0)
        slot = jax.lax.rem(isc, 4)
        grow = gscr[pl.ds(isc, 1), d - 1]
        g = jnp.sum(grow * oh, axis=1, keepdims=True)
        g = jnp.where(valid, g, 0.0)
        feat = featbuf[pl.ds(slot, 1), pl.ds(i, 1)][0, 0]
        acc = acc + g * feat
    return acc * nin_inv


def _fused_body(x_ref, w_ref, b_ref, aw_ref, ab_ref, out_ref,
                featbuf, pooled, ad_s, na_s, gscr):
    s = pl.program_id(1)
    i = pl.program_id(2)

    def _pool_store(vecs):
        base = jnp.where(i == 0, jnp.zeros((_C, _NB), jnp.float32),
                         pooled[...])
        for g, v in enumerate(vecs):
            base = base + v * _onehot(_IPG * i + g)
        pooled[...] = base

    @pl.when(jnp.logical_and(s == 0, i < _PAIRS))
    def _():
        xb = x_ref[...]
        featbuf[0, pl.ds(_IPG * i, _IPG)] = xb
        _pool_store([jnp.sum(xb[g], axis=1)[:, None] * (1.0 / _NPIX)
                     for g in range(_IPG)])

    @pl.when(jnp.logical_and(jnp.logical_and(s >= 1, s <= 6), i < _PAIRS))
    def _():
        nin_inv = jnp.where(s == 1, 1.0, jnp.where(s == 2, 0.5, 1.0 / 3.0))
        accs = [_gather(s, _IPG * i + g, featbuf, gscr, nin_inv)
                for g in range(_IPG)]
        patches = jnp.concatenate(
            [jnp.concatenate(
                [a[:, dy * _WP + dx: dy * _WP + dx + _SWIN] for a in accs],
                axis=1)
             for dy in range(3) for dx in range(3)], axis=0)
        wm = w_ref[pl.ds(s - 1, 1)][0]
        out = jnp.dot(wm, patches, preferred_element_type=jnp.float32)
        bb = b_ref[pl.ds(s - 1, 1)][0]
        out = jnp.maximum(out + bb, 0.0)
        lane = jax.lax.broadcasted_iota(jnp.int32, (1, _IPG * _SWIN), 1)
        out = out * ((lane % _WP) < _W).astype(jnp.float32)
        os = [out[:, g * _SWIN:(g + 1) * _SWIN] for g in range(_IPG)]
        _pool_store([jnp.sum(o, axis=1)[:, None] * (1.0 / _NPIX) for o in os])
        slw = jax.lax.rem(s, 4)
        for g, ob in enumerate(os):
            featbuf[pl.ds(slw, 1), pl.ds(_IPG * i + g, 1), :, 0:_SOUT] = \
                jnp.zeros((1, 1, _C, _SOUT), jnp.float32)
            featbuf[pl.ds(slw, 1), pl.ds(_IPG * i + g, 1), :,
                    _SOUT:_SOUT + _SWIN] = ob[None, None]
            featbuf[pl.ds(slw, 1), pl.ds(_IPG * i + g, 1), :,
                    _SOUT + _SWIN:] = \
                jnp.zeros((1, 1, _C, _SPAD - _SOUT - _SWIN), jnp.float32)

    @pl.when(jnp.logical_and(s == 7, i < _PAIRS))
    def _():
        for g in range(_IPG):
            out_ref[g] = _gather(s, _IPG * i + g, featbuf, gscr, 1.0 / 3.0)

    for sc in range(_N_NODES - 1):
        @pl.when(jnp.logical_and(s == sc, i == _PAIRS))
        def _(sc=sc):
            outs = _OUT_NEIS[sc]
            n_out = len(outs)
            if sc == 0:
                ad = jnp.concatenate(
                    [jnp.ones((1, _NB), jnp.float32),
                     jnp.zeros((_N_NODES - 1, _NB), jnp.float32)], axis=0)
                na = ad
            else:
                ad = ad_s[...]
                na = na_s[...]
            if n_out > 1:
                sco = jnp.dot(aw_ref[sc, :n_out], pooled[...],
                              preferred_element_type=jnp.float32) \
                    + ab_ref[sc, :n_out]
                m = jnp.max(sco, axis=0, keepdims=True)
                e = jnp.exp(sco - m)
                p = e / jnp.sum(e, axis=0, keepdims=True)
            else:
                p = jnp.ones((1, _NB), jnp.float32)
            a_sent = na[sc:sc + 1, :] * p

            rows = []
            for t in range(_N_NODES):
                if t == sc:
                    rows.append(jnp.zeros((1, _NB), jnp.float32))
                elif t in outs:
                    j = outs.index(t)
                    rows.append(ad[t:t + 1] + a_sent[j:j + 1])
                else:
                    rows.append(ad[t:t + 1])
            ad2 = jnp.concatenate(rows, axis=0)

            cnt_rows = []
            for j in range(_N_NODES):
                c = jnp.zeros((1, _NB), jnp.float32)
                for k in range(_N_NODES):
                    if k != j:
                        c = c + (ad2[k:k + 1] > ad2[j:j + 1]).astype(jnp.float32)
                cnt_rows.append(c)
            cnt = jnp.concatenate(cnt_rows, axis=0)
            cand = jnp.where(cnt <= float(_TOPK - 1), ad2, jnp.float32(1e30))
            thresh = jnp.min(cand, axis=0, keepdims=True)

            mask = jnp.logical_and(ad2 >= thresh, ad2 > _EPS).astype(jnp.float32)
            masked = ad2 * mask
            scale = 1.0 / (jnp.sum(masked, axis=0, keepdims=True) + 1e-6)
            na_s[...] = masked * scale
            ad_s[...] = masked

            grows = [a_sent[j:j + 1] * mask[t:t + 1] * scale
                     for j, t in enumerate(outs)]
            grows += [jnp.zeros((1, _NB), jnp.float32)] * (3 - n_out)
            gscr[sc] = jnp.concatenate(grows, axis=0)


def kernel(x, convW, convb, attW, attb):
    B, C, H, W = x.shape
    xl = jnp.pad(x, ((0, 0), (0, 0), (1, _HP - H - 1), (1, _WP - W - 1))
                 ).reshape(B, C, _SPAD)
    wall = jnp.transpose(convW.reshape(6, 9, C, C), (0, 3, 1, 2)
                         ).reshape(6, C, 9 * C)
    bcol = convb.reshape(6, C, 1)
    awT = jnp.transpose(attW, (0, 2, 1))
    abT = attb.reshape(6, 3, 1)

    out = pl.pallas_call(
        _fused_body,
        grid=(_NCORES, _N_NODES, _PAIRS + 1),
        in_specs=[
            pl.BlockSpec(
                (_IPG, C, _SPAD),
                lambda p, s, i: (p * _PAIRS + jnp.where(
                    s == 0, jnp.minimum(i, _PAIRS - 1), _PAIRS - 1), 0, 0)),
            pl.BlockSpec((6, C, 9 * C), lambda p, s, i: (0, 0, 0)),
            pl.BlockSpec((6, C, 1), lambda p, s, i: (0, 0, 0)),
            pl.BlockSpec((6, 3, C), lambda p, s, i: (0, 0, 0)),
            pl.BlockSpec((6, 3, 1), lambda p, s, i: (0, 0, 0)),
        ],
        out_specs=pl.BlockSpec(
            (_IPG, C, _SPAD),
            lambda p, s, i: (p * _PAIRS + jnp.where(
                s == 7, jnp.minimum(i, _PAIRS - 1), 0), 0, 0)),
        out_shape=jax.ShapeDtypeStruct((B, C, _SPAD), jnp.float32),
        scratch_shapes=[
            pltpu.VMEM((4, _NB, C, _SPAD), jnp.float32),
            pltpu.VMEM((C, _NB), jnp.float32),
            pltpu.VMEM((_N_NODES, _NB), jnp.float32),
            pltpu.VMEM((_N_NODES, _NB), jnp.float32),
            pltpu.VMEM((_N_NODES - 1, 3, _NB), jnp.float32),
        ],
        compiler_params=pltpu.CompilerParams(
            dimension_semantics=("parallel", "arbitrary", "arbitrary")),
    )(xl, wall, bcol, awT, abT)
    return out.reshape(B, C, _HP, _WP)[:, :, 1:_H + 1, 1:_W + 1]


# speedup vs baseline: 1.6255x; 1.0099x over previous
import jax
import jax.numpy as jnp
from jax.experimental import pallas as pl
from jax.experimental.pallas import tpu as pltpu

_N_NODES = 8
_OUT_NEIS = [list(range(i + 1, min(i + 4, _N_NODES))) for i in range(_N_NODES - 1)]
_TOPK = 4
_EPS = 0.01

_B = 32
_C = 96
_NCORES = 2
_NB = _B // _NCORES
_IPG = 8
_PAIRS = _NB // _IPG

_H = 28
_W = 28
_HP = 31
_WP = 32
_SPAD = _HP * _WP
_SWIN = _H * _WP
_SOUT = _WP + 1
_NPIX = float(_H * _W)


def _onehot(i):
    lane = jax.lax.broadcasted_iota(jnp.int32, (1, _NB), 1)
    return (lane == i).astype(jnp.float32)


def _gather(s, i, featbuf, gscr, nin_inv):
    oh = _onehot(i)
    acc = jnp.zeros((_C, _SPAD), jnp.float32)
    for d in (1, 2, 3):
        isrc = s - d
        valid = isrc >= 0
        isc = jnp.maximum(isrc, 0)
        slot = jax.lax.rem(isc, 4)
        grow = gscr[pl.ds(isc, 1), d - 1]
        g = jnp.sum(grow * oh, axis=1, keepdims=True)
        g = jnp.where(valid, g, 0.0)
        feat = featbuf[pl.ds(slot, 1), pl.ds(i, 1)][0, 0]
        acc = acc + g * feat
    return acc * nin_inv


def _fused_body(x_ref, w_ref, b_ref, aw_ref, ab_ref, out_ref,
                featbuf, pooled, ad_s, na_s, gscr):
    s = pl.program_id(1)
    i = pl.program_id(2)

    def _pool_store(vecs):
        base = jnp.where(i == 0, jnp.zeros((_C, _NB), jnp.float32),
                         pooled[...])
        for g, v in enumerate(vecs):
            base = base + v * _onehot(_IPG * i + g)
        pooled[...] = base

    @pl.when(jnp.logical_and(s == 0, i < _PAIRS))
    def _():
        xb = x_ref[...]
        featbuf[0, pl.ds(_IPG * i, _IPG)] = xb
        _pool_store([jnp.sum(xb[g], axis=1)[:, None] * (1.0 / _NPIX)
                     for g in range(_IPG)])

    @pl.when(jnp.logical_and(jnp.logical_and(s >= 1, s <= 6), i < _PAIRS))
    def _():
        nin_inv = jnp.where(s == 1, 1.0, jnp.where(s == 2, 0.5, 1.0 / 3.0))
        accs = [_gather(s, _IPG * i + g, featbuf, gscr, nin_inv)
                for g in range(_IPG)]
        patches = jnp.concatenate(
            [jnp.concatenate(
                [a[:, dy * _WP + dx: dy * _WP + dx + _SWIN] for a in accs],
                axis=1)
             for dy in range(3) for dx in range(3)], axis=0)
        wm = w_ref[pl.ds(s - 1, 1)][0]
        out = jnp.dot(wm, patches, preferred_element_type=jnp.float32)
        bb = b_ref[pl.ds(s - 1, 1)][0]
        out = jnp.maximum(out + bb, 0.0)
        lane = jax.lax.broadcasted_iota(jnp.int32, (1, _IPG * _SWIN), 1)
        out = out * ((lane % _WP) < _W).astype(jnp.float32)
        os = [out[:, g * _SWIN:(g + 1) * _SWIN] for g in range(_IPG)]
        _pool_store([jnp.sum(o, axis=1)[:, None] * (1.0 / _NPIX) for o in os])
        slw = jax.lax.rem(s, 4)
        for g, ob in enumerate(os):
            featbuf[pl.ds(slw, 1), pl.ds(_IPG * i + g, 1), :, 0:_SOUT] = \
                jnp.zeros((1, 1, _C, _SOUT), jnp.float32)
            featbuf[pl.ds(slw, 1), pl.ds(_IPG * i + g, 1), :,
                    _SOUT:_SOUT + _SWIN] = ob[None, None]
            featbuf[pl.ds(slw, 1), pl.ds(_IPG * i + g, 1), :,
                    _SOUT + _SWIN:] = \
                jnp.zeros((1, 1, _C, _SPAD - _SOUT - _SWIN), jnp.float32)

    @pl.when(jnp.logical_and(s == 7, i < _PAIRS))
    def _():
        for g in range(_IPG):
            out_ref[g] = _gather(s, _IPG * i + g, featbuf, gscr, 1.0 / 3.0)

    for sc in range(_N_NODES - 1):
        @pl.when(jnp.logical_and(s == sc, i == _PAIRS - 1))
        def _(sc=sc):
            outs = _OUT_NEIS[sc]
            n_out = len(outs)
            if sc == 0:
                ad = jnp.concatenate(
                    [jnp.ones((1, _NB), jnp.float32),
                     jnp.zeros((_N_NODES - 1, _NB), jnp.float32)], axis=0)
                na = ad
            else:
                ad = ad_s[...]
                na = na_s[...]
            if n_out > 1:
                sco = jnp.dot(aw_ref[sc, :n_out], pooled[...],
                              preferred_element_type=jnp.float32) \
                    + ab_ref[sc, :n_out]
                m = jnp.max(sco, axis=0, keepdims=True)
                e = jnp.exp(sco - m)
                p = e / jnp.sum(e, axis=0, keepdims=True)
            else:
                p = jnp.ones((1, _NB), jnp.float32)
            a_sent = na[sc:sc + 1, :] * p

            rows = []
            for t in range(_N_NODES):
                if t == sc:
                    rows.append(jnp.zeros((1, _NB), jnp.float32))
                elif t in outs:
                    j = outs.index(t)
                    rows.append(ad[t:t + 1] + a_sent[j:j + 1])
                else:
                    rows.append(ad[t:t + 1])
            ad2 = jnp.concatenate(rows, axis=0)

            cnt_rows = []
            for j in range(_N_NODES):
                c = jnp.zeros((1, _NB), jnp.float32)
                for k in range(_N_NODES):
                    if k != j:
                        c = c + (ad2[k:k + 1] > ad2[j:j + 1]).astype(jnp.float32)
                cnt_rows.append(c)
            cnt = jnp.concatenate(cnt_rows, axis=0)
            cand = jnp.where(cnt <= float(_TOPK - 1), ad2, jnp.float32(1e30))
            thresh = jnp.min(cand, axis=0, keepdims=True)

            mask = jnp.logical_and(ad2 >= thresh, ad2 > _EPS).astype(jnp.float32)
            masked = ad2 * mask
            scale = 1.0 / (jnp.sum(masked, axis=0, keepdims=True) + 1e-6)
            na_s[...] = masked * scale
            ad_s[...] = masked

            grows = [a_sent[j:j + 1] * mask[t:t + 1] * scale
                     for j, t in enumerate(outs)]
            grows += [jnp.zeros((1, _NB), jnp.float32)] * (3 - n_out)
            gscr[sc] = jnp.concatenate(grows, axis=0)


def kernel(x, convW, convb, attW, attb):
    B, C, H, W = x.shape
    xl = jnp.pad(x, ((0, 0), (0, 0), (1, _HP - H - 1), (1, _WP - W - 1))
                 ).reshape(B, C, _SPAD)
    wall = jnp.transpose(convW.reshape(6, 9, C, C), (0, 3, 1, 2)
                         ).reshape(6, C, 9 * C)
    bcol = convb.reshape(6, C, 1)
    awT = jnp.transpose(attW, (0, 2, 1))
    abT = attb.reshape(6, 3, 1)

    out = pl.pallas_call(
        _fused_body,
        grid=(_NCORES, _N_NODES, _PAIRS),
        in_specs=[
            pl.BlockSpec(
                (_IPG, C, _SPAD),
                lambda p, s, i: (p * _PAIRS + jnp.where(
                    s == 0, jnp.minimum(i, _PAIRS - 1), _PAIRS - 1), 0, 0)),
            pl.BlockSpec((6, C, 9 * C), lambda p, s, i: (0, 0, 0)),
            pl.BlockSpec((6, C, 1), lambda p, s, i: (0, 0, 0)),
            pl.BlockSpec((6, 3, C), lambda p, s, i: (0, 0, 0)),
            pl.BlockSpec((6, 3, 1), lambda p, s, i: (0, 0, 0)),
        ],
        out_specs=pl.BlockSpec(
            (_IPG, C, _SPAD),
            lambda p, s, i: (p * _PAIRS + jnp.where(
                s == 7, jnp.minimum(i, _PAIRS - 1), 0), 0, 0)),
        out_shape=jax.ShapeDtypeStruct((B, C, _SPAD), jnp.float32),
        scratch_shapes=[
            pltpu.VMEM((4, _NB, C, _SPAD), jnp.float32),
            pltpu.VMEM((C, _NB), jnp.float32),
            pltpu.VMEM((_N_NODES, _NB), jnp.float32),
            pltpu.VMEM((_N_NODES, _NB), jnp.float32),
            pltpu.VMEM((_N_NODES - 1, 3, _NB), jnp.float32),
        ],
        compiler_params=pltpu.CompilerParams(
            dimension_semantics=("parallel", "arbitrary", "arbitrary")),
    )(xl, wall, bcol, awT, abT)
    return out.reshape(B, C, _HP, _WP)[:, :, 1:_H + 1, 1:_W + 1]
